# baseline (speedup 1.0000x reference)
import functools

import jax
import jax.numpy as jnp
from jax import lax
from jax.experimental import pallas as pl
from jax.experimental.pallas import tpu as pltpu
from jax.experimental.pallas import tpu_sc as plsc

NC = 2
NS = 16
NW = NC * NS

LANES = 16
NBUF = 4
ROUND_MAGIC = 8388608.0


@functools.lru_cache(maxsize=None)
def _build(b, t, a, vocab, embed_dim):
    n_pairs = t * a
    ppw = n_pairs // NW
    eh_n = embed_dim // 8
    bh_n = b // 128
    assert ppw * NW == n_pairs
    assert eh_n * 8 == embed_dim and bh_n * 128 == b
    assert eh_n % NBUF == 0 and b % LANES == 0
    mesh = plsc.VectorSubcoreMesh(core_axis_name="c", subcore_axis_name="s")

    @functools.partial(
        pl.kernel,
        mesh=mesh,
        compiler_params=pltpu.CompilerParams(use_tc_tiling_on_sc=False, needs_layout_passes=False),
        out_type=jax.ShapeDtypeStruct(
            (t, a, eh_n, bh_n, 8, 128), jnp.float32),
        scratch_types=[
            pltpu.VMEM((vocab * embed_dim,), jnp.float32),
            pltpu.VMEM((ppw * b,), jnp.float32),
            pltpu.VMEM((b,), jnp.int32),
            pltpu.VMEM((NBUF, bh_n, 8, 128), jnp.float32),
            pltpu.SemaphoreType.DMA,
        ],
    )
    def k(act_hbm, tab_hbm, out_hbm, tab_v, act_v, tok_v, bufs, sem_s):
        wid = lax.axis_index("s") * NC + lax.axis_index("c")
        p0 = wid * ppw
        pltpu.sync_copy(tab_hbm, tab_v)
        pltpu.sync_copy(act_hbm.at[pl.ds(p0 * b, ppw * b)], act_v)

        def wait_one_scatter():
            pltpu.make_async_copy(
                bufs.at[0], out_hbm.at[0, 0, 0], sem_s).wait()

        def pair_body(q, _):
            p = p0 + q
            tt = lax.div(p, a)
            aa = lax.rem(p, a)

            @plsc.parallel_loop(0, b // LANES, unroll=4)
            def quant(i):
                x = act_v[pl.ds(q * b + i * LANES, LANES)]
                s = (x - (-1.0)) / 2.0 * (vocab - 1.0)
                v = (s + ROUND_MAGIC) - ROUND_MAGIC
                v = jnp.minimum(jnp.maximum(v, 0.0), vocab - 1.0)
                tok_v[pl.ds(i * LANES, LANES)] = v.astype(jnp.int32)

            for eh in range(eh_n):
                buf = bufs.at[eh % NBUF]
                if eh < NBUF:
                    @pl.when(q > 0)
                    def _():
                        wait_one_scatter()
                else:
                    wait_one_scatter()

                @plsc.parallel_loop(0, bh_n * 8, unroll=4)
                def unit(g):
                    bh = lax.div(g, 8)
                    gg = lax.rem(g, 8)
                    tok16 = tok_v[pl.ds(g * LANES, LANES)]
                    for el in range(8):
                        idx = tok16 + (eh * 8 + el) * vocab
                        val = plsc.load_gather(tab_v, [idx])
                        buf[bh, el, pl.ds(gg * LANES, LANES)] = val
                pltpu.async_copy(buf, out_hbm.at[tt, aa, eh], sem_s)
            return 0

        lax.fori_loop(0, ppw, pair_body, 0)
        for _ in range(NBUF):
            wait_one_scatter()

    return k


def kernel(actions, embedding):
    b, t, a = actions.shape
    vocab, embed_dim = embedding.shape
    act_flat = actions.transpose(1, 2, 0).reshape(b * t * a)
    tab_flat = embedding.T.reshape(vocab * embed_dim)
    o = _build(b, t, a, vocab, embed_dim)(act_flat, tab_flat)
    o = o.transpose(3, 5, 0, 1, 2, 4)
    return o.reshape(b, t, a, embed_dim)
